# baseline (speedup 1.0000x reference)
import jax
import jax.numpy as jnp
from jax import lax
from jax.experimental import pallas as pl
from jax.experimental.pallas import tpu as pltpu
from jax.experimental.pallas import tpu_sc as plsc


def kernel(z_base_global, domain_ids):
    del domain_ids
    rows, cols = z_base_global.shape
    info = plsc.get_sparse_core_info()
    nc, ns = info.num_cores, info.num_subcores
    nw = nc * ns
    rows_per_w = rows // nw
    mesh = plsc.VectorSubcoreMesh(core_axis_name="c", subcore_axis_name="s")

    @jax.named_call
    def _run(z):
        def body(z_hbm, out_hbm, sem):
            wid = lax.axis_index("s") * nc + lax.axis_index("c")
            base = wid * rows_per_w
            pltpu.async_copy(
                z_hbm.at[pl.ds(base, rows_per_w), :],
                out_hbm.at[pl.ds(base, rows_per_w), :],
                sem,
            ).wait()

        return pl.kernel(
            body,
            mesh=mesh,
            out_type=jax.ShapeDtypeStruct((rows, cols), z_base_global.dtype),
            scratch_types=[pltpu.SemaphoreType.DMA],
        )(z)

    return _run(z_base_global)


# speedup vs baseline: 23.3403x; 23.3403x over previous
import jax
import jax.numpy as jnp
from jax import lax
from jax.experimental import pallas as pl
from jax.experimental.pallas import tpu as pltpu
from jax.experimental.pallas import tpu_sc as plsc

_CHUNK_ROWS = 64


def kernel(z_base_global, domain_ids):
    del domain_ids
    rows, cols = z_base_global.shape
    info = plsc.get_sparse_core_info()
    nc, ns = info.num_cores, info.num_subcores
    nw = nc * ns
    rows_per_w = rows // nw
    nchunks = rows_per_w // _CHUNK_ROWS
    mesh = plsc.VectorSubcoreMesh(core_axis_name="c", subcore_axis_name="s")

    def body(z_hbm, out_hbm, buf0, buf1, rsem, wsem):
        wid = lax.axis_index("s") * nc + lax.axis_index("c")
        base = wid * rows_per_w
        bufs = (buf0, buf1)
        writes = [None] * nchunks
        reads = [None] * nchunks
        for j in range(nchunks):
            if j >= 2:
                writes[j - 2].wait()
            reads[j] = pltpu.async_copy(
                z_hbm.at[pl.ds(base + j * _CHUNK_ROWS, _CHUNK_ROWS), :],
                bufs[j % 2],
                rsem,
            )
            reads[j].wait()
            writes[j] = pltpu.async_copy(
                bufs[j % 2],
                out_hbm.at[pl.ds(base + j * _CHUNK_ROWS, _CHUNK_ROWS), :],
                wsem,
            )
        writes[nchunks - 2].wait()
        writes[nchunks - 1].wait()

    return pl.kernel(
        body,
        mesh=mesh,
        out_type=jax.ShapeDtypeStruct((rows, cols), z_base_global.dtype),
        scratch_types=[
            pltpu.VMEM((_CHUNK_ROWS, cols), z_base_global.dtype),
            pltpu.VMEM((_CHUNK_ROWS, cols), z_base_global.dtype),
            pltpu.SemaphoreType.DMA,
            pltpu.SemaphoreType.DMA,
        ],
    )(z_base_global)


# speedup vs baseline: 50.0299x; 2.1435x over previous
import jax
import jax.numpy as jnp
from jax.experimental import pallas as pl
from jax.experimental.pallas import tpu as pltpu

_BLOCK_ROWS = 4096


def _copy_block(z_ref, o_ref):
    o_ref[...] = z_ref[...]


def kernel(z_base_global, domain_ids):
    del domain_ids
    rows, cols = z_base_global.shape
    grid = (rows // _BLOCK_ROWS,)
    return pl.pallas_call(
        _copy_block,
        grid=grid,
        in_specs=[pl.BlockSpec((_BLOCK_ROWS, cols), lambda i: (i, 0))],
        out_specs=pl.BlockSpec((_BLOCK_ROWS, cols), lambda i: (i, 0)),
        out_shape=jax.ShapeDtypeStruct((rows, cols), z_base_global.dtype),
        compiler_params=pltpu.CompilerParams(
            dimension_semantics=("parallel",),
        ),
    )(z_base_global)


# speedup vs baseline: 50.7983x; 1.0154x over previous
import jax
import jax.numpy as jnp
from jax.experimental import pallas as pl
from jax.experimental.pallas import tpu as pltpu

_CHUNKS = (1024, 2048, 4096, 4096, 4096, 1024)


def _copy_manual(z_ref, o_ref, buf, rsem, wsem):
    n = len(_CHUNKS)
    bases = [sum(_CHUNKS[:i]) for i in range(n)]
    reads = [
        pltpu.make_async_copy(
            z_ref.at[pl.ds(bases[i], _CHUNKS[i]), :],
            buf.at[pl.ds(bases[i], _CHUNKS[i]), :],
            rsem.at[i],
        )
        for i in range(n)
    ]
    writes = [
        pltpu.make_async_copy(
            buf.at[pl.ds(bases[i], _CHUNKS[i]), :],
            o_ref.at[pl.ds(bases[i], _CHUNKS[i]), :],
            wsem.at[i],
        )
        for i in range(n)
    ]
    for r in reads:
        r.start()
    for i in range(n):
        reads[i].wait()
        writes[i].start()
    for w in writes:
        w.wait()


def kernel(z_base_global, domain_ids):
    del domain_ids
    rows, cols = z_base_global.shape
    return pl.pallas_call(
        _copy_manual,
        in_specs=[pl.BlockSpec(memory_space=pl.ANY)],
        out_specs=pl.BlockSpec(memory_space=pl.ANY),
        out_shape=jax.ShapeDtypeStruct((rows, cols), z_base_global.dtype),
        scratch_shapes=[
            pltpu.VMEM((rows, cols), z_base_global.dtype),
            pltpu.SemaphoreType.DMA((len(_CHUNKS),)),
            pltpu.SemaphoreType.DMA((len(_CHUNKS),)),
        ],
    )(z_base_global)


# speedup vs baseline: 50.9459x; 1.0029x over previous
import jax
import jax.numpy as jnp
from jax.experimental import pallas as pl
from jax.experimental.pallas import tpu as pltpu

_CHUNKS = (512, 1024, 2048, 4096, 4096, 2048, 1024, 1024, 512)


def _copy_manual(z_ref, o_ref, buf, rsem, wsem):
    n = len(_CHUNKS)
    bases = [sum(_CHUNKS[:i]) for i in range(n)]
    reads = [
        pltpu.make_async_copy(
            z_ref.at[pl.ds(bases[i], _CHUNKS[i]), :],
            buf.at[pl.ds(bases[i], _CHUNKS[i]), :],
            rsem.at[i],
        )
        for i in range(n)
    ]
    writes = [
        pltpu.make_async_copy(
            buf.at[pl.ds(bases[i], _CHUNKS[i]), :],
            o_ref.at[pl.ds(bases[i], _CHUNKS[i]), :],
            wsem.at[i],
        )
        for i in range(n)
    ]
    for r in reads:
        r.start()
    for i in range(n):
        reads[i].wait()
        writes[i].start()
    for w in writes:
        w.wait()


def kernel(z_base_global, domain_ids):
    del domain_ids
    rows, cols = z_base_global.shape
    return pl.pallas_call(
        _copy_manual,
        in_specs=[pl.BlockSpec(memory_space=pl.ANY)],
        out_specs=pl.BlockSpec(memory_space=pl.ANY),
        out_shape=jax.ShapeDtypeStruct((rows, cols), z_base_global.dtype),
        scratch_shapes=[
            pltpu.VMEM((rows, cols), z_base_global.dtype),
            pltpu.SemaphoreType.DMA((len(_CHUNKS),)),
            pltpu.SemaphoreType.DMA((len(_CHUNKS),)),
        ],
    )(z_base_global)
